# Initial kernel scaffold; baseline (speedup 1.0000x reference)
import jax
import jax.numpy as jnp
from jax.experimental import pallas as pl


def kernel(x_0, incidence_1, W1, b1, W2, b2):
    raise NotImplementedError("write your pallas kernel here")


# speedup vs baseline: 1.5000x; 1.5000x over previous
import jax
import jax.numpy as jnp
from jax.experimental import pallas as pl
from jax.experimental.pallas import tpu as pltpu

N_NODES = 10000
N_EDGES = 2048
D = 128
TILE = 400
T = N_NODES // TILE

_TN = (((0,), (0,)), ((), ()))


def _uni_kernel(x0_ref, B_ref, W1_ref, b1_ref, W2_ref, b2_ref,
                x0_out_ref, x1_out_ref,
                Bbf, h1s, h2s, acc1, acc2, degs):
    p = pl.program_id(0)
    t = pl.program_id(1)
    rows = pl.ds(t * TILE, TILE)

    @pl.when(p == 0)
    def _phase_a():
        Bi = B_ref[...].astype(jnp.bfloat16)
        Bbf[rows, :] = Bi
        h1 = jnp.dot(x0_ref[...].astype(jnp.bfloat16),
                     W1_ref[...].astype(jnp.bfloat16),
                     preferred_element_type=jnp.float32) + b1_ref[...]
        h1b = h1.astype(jnp.bfloat16)
        h1s[rows, :] = h1b
        contrib = jax.lax.dot_general(Bi, h1b, _TN,
                                      preferred_element_type=jnp.float32)
        ones = jnp.ones((TILE, 8), jnp.bfloat16)
        dcon = jax.lax.dot_general(Bi, ones, _TN,
                                   preferred_element_type=jnp.float32)

        @pl.when(t == 0)
        def _():
            acc1[...] = contrib
            degs[...] = dcon

        @pl.when(t != 0)
        def _():
            acc1[...] += contrib
            degs[...] += dcon

    @pl.when(p == 1)
    def _phase_b():
        @pl.when(t == 0)
        def _():
            deg = degs[:, 0:1]
            acc1[...] = acc1[...] / jnp.where(deg == 0.0, 1.0, deg)

        Bi = Bbf[rows, :]
        agg = jnp.dot(Bi, acc1[...].astype(jnp.bfloat16),
                      preferred_element_type=jnp.float32)
        x0l1 = jnp.maximum(h1s[rows, :].astype(jnp.float32) + agg, 0.0)
        h2 = jnp.dot(x0l1.astype(jnp.bfloat16),
                     W2_ref[...].astype(jnp.bfloat16),
                     preferred_element_type=jnp.float32) + b2_ref[...]
        h2b = h2.astype(jnp.bfloat16)
        h2s[rows, :] = h2b
        contrib = jax.lax.dot_general(Bi, h2b, _TN,
                                      preferred_element_type=jnp.float32)

        @pl.when(t == 0)
        def _():
            acc2[...] = contrib

        @pl.when(t != 0)
        def _():
            acc2[...] += contrib

    @pl.when(p == 2)
    def _phase_c():
        @pl.when(t == 0)
        def _():
            deg = degs[:, 0:1]
            x1_2 = acc2[...] / jnp.where(deg == 0.0, 1.0, deg)
            acc2[...] = x1_2
            x1_out_ref[...] = x1_2

        Bi = Bbf[rows, :]
        agg = jnp.dot(Bi, acc2[...].astype(jnp.bfloat16),
                      preferred_element_type=jnp.float32)
        x0_out_ref[...] = jnp.maximum(
            h2s[rows, :].astype(jnp.float32) + agg, 0.0)


def _run(x_0, incidence_1, W1, b1, W2, b2, interpret=False):
    return pl.pallas_call(
        _uni_kernel,
        grid=(3, T),
        in_specs=[
            pl.BlockSpec((TILE, D), lambda p, t: (jnp.where(p == 0, t, 0), 0)),
            pl.BlockSpec((TILE, N_EDGES),
                         lambda p, t: (jnp.where(p == 0, t, 0), 0)),
            pl.BlockSpec((D, D), lambda p, t: (0, 0)),
            pl.BlockSpec((1, D), lambda p, t: (0, 0)),
            pl.BlockSpec((D, D), lambda p, t: (0, 0)),
            pl.BlockSpec((1, D), lambda p, t: (0, 0)),
        ],
        out_specs=[
            pl.BlockSpec((TILE, D), lambda p, t: (jnp.where(p == 2, t, 0), 0)),
            pl.BlockSpec((N_EDGES, D), lambda p, t: (0, 0)),
        ],
        out_shape=[
            jax.ShapeDtypeStruct((N_NODES, D), jnp.float32),
            jax.ShapeDtypeStruct((N_EDGES, D), jnp.float32),
        ],
        scratch_shapes=[
            pltpu.VMEM((N_NODES, N_EDGES), jnp.bfloat16),
            pltpu.VMEM((N_NODES, D), jnp.bfloat16),
            pltpu.VMEM((N_NODES, D), jnp.bfloat16),
            pltpu.VMEM((N_EDGES, D), jnp.float32),
            pltpu.VMEM((N_EDGES, D), jnp.float32),
            pltpu.VMEM((N_EDGES, 8), jnp.float32),
        ],
        interpret=interpret,
    )(x_0, incidence_1, W1, b1.reshape(1, D), W2, b2.reshape(1, D))


def kernel(x_0, incidence_1, W1, b1, W2, b2):
    x0_out, x1_out = _run(x_0, incidence_1, W1, b1, W2, b2)
    return (x0_out, x1_out)


# speedup vs baseline: 1.6425x; 1.0950x over previous
import jax
import jax.numpy as jnp
from jax.experimental import pallas as pl
from jax.experimental.pallas import tpu as pltpu

N_NODES = 10000
N_EDGES = 2048
D = 128
TILE = 400
T = N_NODES // TILE

_TN = (((0,), (0,)), ((), ()))


def _uni_kernel(x0_ref, B_ref, W1_ref, b1_ref, W2_ref, b2_ref,
                x0_out_ref, x1_out_ref,
                Bbf, h1s, h2s, accT, x1b, degsT):
    p = pl.program_id(0)
    t = pl.program_id(1)
    rows = pl.ds(t * TILE, TILE)

    @pl.when(p == 0)
    def _phase_a():
        Bi = B_ref[...].astype(jnp.bfloat16)
        Bbf[rows, :] = Bi
        h1 = jnp.dot(x0_ref[...].astype(jnp.bfloat16),
                     W1_ref[...].astype(jnp.bfloat16),
                     preferred_element_type=jnp.float32) + b1_ref[...]
        h1b = h1.astype(jnp.bfloat16)
        h1s[rows, :] = h1b
        contribT = jax.lax.dot_general(h1b, Bi, _TN,
                                       preferred_element_type=jnp.float32)
        ones = jnp.ones((TILE, 8), jnp.bfloat16)
        dconT = jax.lax.dot_general(ones, Bi, _TN,
                                    preferred_element_type=jnp.float32)

        @pl.when(t == 0)
        def _():
            accT[...] = contribT
            degsT[...] = dconT

        @pl.when(t != 0)
        def _():
            accT[...] += contribT
            degsT[...] += dconT

    @pl.when(p == 1)
    def _phase_b():
        @pl.when(t == 0)
        def _():
            deg = degsT[0:1, :]
            x1T = accT[...] / jnp.where(deg == 0.0, 1.0, deg)
            x1b[...] = jnp.transpose(x1T.astype(jnp.bfloat16))

        Bi = Bbf[rows, :]
        agg = jnp.dot(Bi, x1b[...], preferred_element_type=jnp.float32)
        x0l1 = jnp.maximum(h1s[rows, :].astype(jnp.float32) + agg, 0.0)
        h2 = jnp.dot(x0l1.astype(jnp.bfloat16),
                     W2_ref[...].astype(jnp.bfloat16),
                     preferred_element_type=jnp.float32) + b2_ref[...]
        h2b = h2.astype(jnp.bfloat16)
        h2s[rows, :] = h2b
        contribT = jax.lax.dot_general(h2b, Bi, _TN,
                                       preferred_element_type=jnp.float32)

        @pl.when(t == 0)
        def _():
            accT[...] = contribT

        @pl.when(t != 0)
        def _():
            accT[...] += contribT

    @pl.when(p == 2)
    def _phase_c():
        @pl.when(t == 0)
        def _():
            deg = degsT[0:1, :]
            x1T = accT[...] / jnp.where(deg == 0.0, 1.0, deg)
            x1b[...] = jnp.transpose(x1T.astype(jnp.bfloat16))
            x1_out_ref[...] = jnp.transpose(x1T)

        Bi = Bbf[rows, :]
        agg = jnp.dot(Bi, x1b[...], preferred_element_type=jnp.float32)
        x0_out_ref[...] = jnp.maximum(
            h2s[rows, :].astype(jnp.float32) + agg, 0.0)


def _run(x_0, incidence_1, W1, b1, W2, b2, interpret=False):
    return pl.pallas_call(
        _uni_kernel,
        grid=(3, T),
        in_specs=[
            pl.BlockSpec((TILE, D), lambda p, t: (jnp.where(p == 0, t, 0), 0)),
            pl.BlockSpec((TILE, N_EDGES),
                         lambda p, t: (jnp.where(p == 0, t, 0), 0)),
            pl.BlockSpec((D, D), lambda p, t: (0, 0)),
            pl.BlockSpec((1, D), lambda p, t: (0, 0)),
            pl.BlockSpec((D, D), lambda p, t: (0, 0)),
            pl.BlockSpec((1, D), lambda p, t: (0, 0)),
        ],
        out_specs=[
            pl.BlockSpec((TILE, D), lambda p, t: (jnp.where(p == 2, t, 0), 0)),
            pl.BlockSpec((N_EDGES, D), lambda p, t: (0, 0)),
        ],
        out_shape=[
            jax.ShapeDtypeStruct((N_NODES, D), jnp.float32),
            jax.ShapeDtypeStruct((N_EDGES, D), jnp.float32),
        ],
        scratch_shapes=[
            pltpu.VMEM((N_NODES, N_EDGES), jnp.bfloat16),
            pltpu.VMEM((N_NODES, D), jnp.bfloat16),
            pltpu.VMEM((N_NODES, D), jnp.bfloat16),
            pltpu.VMEM((D, N_EDGES), jnp.float32),
            pltpu.VMEM((N_EDGES, D), jnp.bfloat16),
            pltpu.VMEM((8, N_EDGES), jnp.float32),
        ],
        interpret=interpret,
    )(x_0, incidence_1, W1, b1.reshape(1, D), W2, b2.reshape(1, D))


def kernel(x_0, incidence_1, W1, b1, W2, b2):
    x0_out, x1_out = _run(x_0, incidence_1, W1, b1, W2, b2)
    return (x0_out, x1_out)
